# baseline (speedup 1.0000x reference)
import functools

import jax
import jax.numpy as jnp
from jax import lax
from jax.experimental import pallas as pl
from jax.experimental.pallas import tpu as pltpu
from jax.experimental.pallas import tpu_sc as plsc

_B = 4096
_N = 20
_D = 32
_P = _N * (_N - 1) // 2
_CHUNK = 128


def _sc_gather(mem_div, user_div, item_packed, user_packed):
    info = plsc.get_sparse_core_info()
    nw = info.num_cores * info.num_subcores
    rpw = _B * _N // nw
    n_chunks = rpw // _CHUNK
    upw = _B // nw

    mesh = plsc.VectorSubcoreMesh(core_axis_name="c", subcore_axis_name="s")

    @functools.partial(
        pl.kernel,
        mesh=mesh,
        out_type=[
            jax.ShapeDtypeStruct((_B * _N, 128), jnp.float32),
            jax.ShapeDtypeStruct((_B, 128), jnp.float32),
        ],
        scratch_types=[
            pltpu.VMEM((n_chunks, _CHUNK), jnp.int32),
            pltpu.VMEM((_CHUNK,), jnp.int32),
            pltpu.VMEM((2, _CHUNK, 128), jnp.float32),
            pltpu.VMEM((upw, 128), jnp.float32),
            pltpu.SemaphoreType.DMA,
            pltpu.SemaphoreType.DMA,
        ],
    )
    def k(mem_div_hbm, user_div_hbm, item_t_hbm, user_t_hbm,
          item_out, user_out, idx_v, uidx_v, buf_v, urows_v, sem, usem):
        wid = lax.axis_index("s") * info.num_cores + lax.axis_index("c")
        rbase = wid * rpw
        pltpu.sync_copy(mem_div_hbm.at[wid], idx_v)
        pltpu.sync_copy(user_div_hbm.at[wid], uidx_v)

        ucopy = pltpu.async_copy(user_t_hbm.at[uidx_v], urows_v, usem)
        copies = [None, None]
        copies[0] = pltpu.async_copy(
            item_t_hbm.at[idx_v.at[0]], buf_v.at[0], sem)
        for c in range(n_chunks):
            if c + 1 < n_chunks:
                copies[(c + 1) % 2] = pltpu.async_copy(
                    item_t_hbm.at[idx_v.at[c + 1]], buf_v.at[(c + 1) % 2], sem)
            copies[c % 2].wait()
            pltpu.sync_copy(
                buf_v.at[c % 2],
                item_out.at[pl.ds(rbase + c * _CHUNK, _CHUNK)])

        ucopy.wait()
        pltpu.sync_copy(urows_v, user_out.at[pl.ds(wid * upw, upw)])

    return k(mem_div, user_div, item_packed, user_packed)


def _extract(packed, colb):
    total = None
    for kq in range(4):
        piece = jnp.where(colb == kq * _D,
                          packed[..., kq * _D:(kq + 1) * _D], 0.0)
        total = piece if total is None else total + piece
    return total


def _fused_body(upref, ucbref, xpref, cbref, wref, oref):
    ue = _extract(upref[...], ucbref[...])
    x3 = xpref[...]
    cb3 = cbref[...]
    w = wref[...]
    we = []
    for n in range(_N):
        yn = _extract(x3[n], cb3[n])
        we.append(yn * w[:, n * _D:(n + 1) * _D])
    parts = [jnp.concatenate([ue * we[n] for n in range(_N)], axis=1)]
    for i in range(_N - 1):
        parts.append(jnp.concatenate(
            [we[i] * we[j] for j in range(i + 1, _N)], axis=1))
    oref[...] = jnp.concatenate(parts, axis=1)


def kernel(user, memory, user_table, item_table, weights):
    nw = 32
    mem_nm = memory.astype(jnp.int32).T
    mem_div = (mem_nm.reshape(-1) >> 2).reshape(nw, -1, _CHUNK)
    cb32 = jnp.broadcast_to(((mem_nm & 3) * _D)[:, :, None], (_N, _B, _D))
    user_idx = user.reshape(-1).astype(jnp.int32)
    user_div = (user_idx >> 2).reshape(nw, _CHUNK)
    ucb32 = jnp.broadcast_to(((user_idx & 3) * _D)[:, None], (_B, _D))

    item_packed = item_table[: (item_table.shape[0] // 4) * 4].reshape(-1, 128)
    user_packed = user_table.reshape(-1, 128)

    item_rows, user_rows = _sc_gather(
        mem_div, user_div, item_packed, user_packed)
    xp = item_rows.reshape(_N, _B, 128)
    wcols = jnp.repeat(weights, _D)[None, :]

    bb = 128
    grid = (_B // bb,)
    out = pl.pallas_call(
        _fused_body,
        grid=grid,
        in_specs=[
            pl.BlockSpec((bb, 128), lambda i: (i, 0)),
            pl.BlockSpec((bb, _D), lambda i: (i, 0)),
            pl.BlockSpec((_N, bb, 128), lambda i: (0, i, 0)),
            pl.BlockSpec((_N, bb, _D), lambda i: (0, i, 0)),
            pl.BlockSpec((1, _N * _D), lambda i: (0, 0)),
        ],
        out_specs=pl.BlockSpec((bb, (_N + _P) * _D), lambda i: (i, 0)),
        out_shape=jax.ShapeDtypeStruct((_B, (_N + _P) * _D), jnp.float32),
    )(user_rows, ucb32, xp, cb32, wcols)
    return out


# speedup vs baseline: 1.0352x; 1.0352x over previous
import functools

import jax
import jax.numpy as jnp
from jax import lax
from jax.experimental import pallas as pl
from jax.experimental.pallas import tpu as pltpu
from jax.experimental.pallas import tpu_sc as plsc

_B = 4096
_N = 20
_D = 32
_P = _N * (_N - 1) // 2
_ROWS = 1000000
_PACK = 128 // _D


def _sc_gather(pidx_item, pidx_user, item_packed, user_packed):
    info = plsc.get_sparse_core_info()
    nw = info.num_cores * info.num_subcores
    bpw = _B // nw
    rpw = bpw * _N
    chunk = 128
    nchunks = rpw // chunk

    mesh = plsc.VectorSubcoreMesh(core_axis_name="c", subcore_axis_name="s")

    @functools.partial(
        pl.kernel,
        mesh=mesh,
        out_type=[
            jax.ShapeDtypeStruct((_B * _N, 128), jnp.float32),
            jax.ShapeDtypeStruct((_B, 128), jnp.float32),
        ],
        scratch_types=[
            pltpu.VMEM((rpw,), jnp.int32),
            pltpu.VMEM((bpw,), jnp.int32),
            pltpu.VMEM((2, chunk, 128), jnp.float32),
            pltpu.VMEM((bpw, 128), jnp.float32),
            pltpu.SemaphoreType.DMA,
            pltpu.SemaphoreType.DMA,
            pltpu.SemaphoreType.DMA,
        ],
    )
    def k(mem_idx_hbm, user_idx_hbm, item_t_hbm, user_t_hbm,
          item_out, user_out, vidx, vuidx, vrows, vurows, gsem0, gsem1, usem):
        wid = lax.axis_index("s") * info.num_cores + lax.axis_index("c")
        rbase = wid * rpw
        bbase = wid * bpw
        pltpu.sync_copy(mem_idx_hbm.at[pl.ds(rbase, rpw)], vidx)
        pltpu.sync_copy(user_idx_hbm.at[pl.ds(bbase, bpw)], vuidx)

        ucopy = pltpu.make_async_copy(
            user_t_hbm.at[vuidx], vurows, usem)
        ucopy.start()

        gsems = (gsem0, gsem1)
        copies = []
        for g in range(nchunks):
            c = pltpu.make_async_copy(
                item_t_hbm.at[vidx.at[pl.ds(g * chunk, chunk)]],
                vrows.at[g % 2], gsems[g % 2])
            if g >= 2:
                copies[g - 2].wait()
            c.start()
            c.wait()
            oc = pltpu.make_async_copy(
                vrows.at[g % 2],
                item_out.at[pl.ds(rbase + g * chunk, chunk)],
                gsems[g % 2])
            oc.start()
            copies.append(oc)
        copies[nchunks - 2].wait()
        copies[nchunks - 1].wait()

        ucopy.wait()
        pltpu.sync_copy(vurows, user_out.at[pl.ds(bbase, bpw)])

    return k(pidx_item, pidx_user, item_packed, user_packed)


def _extract(packed, offs, n):
    res = None
    for kk in range(_PACK):
        sel = jnp.concatenate(
            [packed[:, j * 128 + kk * _D:j * 128 + (kk + 1) * _D]
             for j in range(n)], axis=1)
        cur = jnp.where(offs == kk, sel, 0.0)
        res = cur if res is None else res + cur
    return res


def _expand_body(upref, ipref, uoref, ioref, wref, oref):
    ue = _extract(upref[...], uoref[...], 1)
    ie = _extract(ipref[...], ioref[...], _N)
    we = ie * wref[...]
    parts = [jnp.concatenate([ue] * _N, axis=1) * we]
    for i in range(_N - 1):
        li = we[:, i * _D:(i + 1) * _D]
        rep = _N - 1 - i
        parts.append(jnp.concatenate([li] * rep, axis=1) * we[:, (i + 1) * _D:])
    oref[...] = jnp.concatenate(parts, axis=1)


def kernel(user, memory, user_table, item_table, weights):
    mem_idx = memory.reshape(-1).astype(jnp.int32)
    user_idx = user.reshape(-1).astype(jnp.int32)
    item_packed = item_table[:_ROWS].reshape(_ROWS // _PACK, 128)
    user_packed = user_table.reshape(_ROWS // _PACK, 128)

    item_rows, user_rows = _sc_gather(
        mem_idx // _PACK, user_idx // _PACK, item_packed, user_packed)
    ioffs = jnp.repeat(memory.astype(jnp.int32) % _PACK, _D, axis=1)
    uoffs = jnp.repeat(user.astype(jnp.int32) % _PACK, _D, axis=1)
    wcols = jnp.repeat(weights, _D)[None, :]

    item_2d = item_rows.reshape(_B, _N * 128)

    bb = 128
    grid = (_B // bb,)
    out = pl.pallas_call(
        _expand_body,
        grid=grid,
        in_specs=[
            pl.BlockSpec((bb, 128), lambda i: (i, 0)),
            pl.BlockSpec((bb, _N * 128), lambda i: (i, 0)),
            pl.BlockSpec((bb, _D), lambda i: (i, 0)),
            pl.BlockSpec((bb, _N * _D), lambda i: (i, 0)),
            pl.BlockSpec((1, _N * _D), lambda i: (0, 0)),
        ],
        out_specs=pl.BlockSpec((bb, (_N + _P) * _D), lambda i: (i, 0)),
        out_shape=jax.ShapeDtypeStruct((_B, (_N + _P) * _D), jnp.float32),
    )(user_rows, item_2d, uoffs, ioffs, wcols)
    return out
